# Initial kernel scaffold; baseline (speedup 1.0000x reference)
import jax
import jax.numpy as jnp
from jax.experimental import pallas as pl


def kernel(pos_u, pos_v, neg_v, u_weight, v_weight):
    raise NotImplementedError("write your pallas kernel here")


# speedup vs baseline: 4.8049x; 4.8049x over previous
import functools

import jax
import jax.numpy as jnp
from jax import lax
from jax.experimental import pallas as pl
from jax.experimental.pallas import tpu as pltpu
from jax.experimental.pallas import tpu_sc as plsc

B = 16384
D = 64
T = 25
NPOS = 5
NC = 2
NS = 16
NW = NC * NS
BW = B // NW
W = 32
CH = BW // W
WT = W * T
LANES = 16

_GCHUNK = 128


def _sc_scores_body(pos_u_hbm, tgt_hbm, u_w_hbm, v_w_hbm, out_hbm,
                    uidx, tidx, urows, vrows, pbuf, sem):
    wid = lax.axis_index("s") * NC + lax.axis_index("c")
    base = wid * BW

    @pl.loop(0, CH)
    def _chunk(ch):
        b0 = base + ch * W
        pltpu.sync_copy(pos_u_hbm.at[pl.ds(b0, W)], uidx)
        pltpu.sync_copy(tgt_hbm.at[pl.ds(b0 * T, WT)], tidx)

        copies = [pltpu.async_copy(u_w_hbm.at[uidx], urows, sem)]
        off = 0
        while off < WT:
            n = min(_GCHUNK, WT - off)
            copies.append(pltpu.async_copy(
                v_w_hbm.at[tidx.at[pl.ds(off, n)]],
                vrows.at[pl.ds(off, n)], sem))
            off += n
        for cp in copies:
            cp.wait()

        @pl.loop(0, W)
        def _elem(b):
            u0 = urows[b, pl.ds(0, LANES)]
            u1 = urows[b, pl.ds(LANES, LANES)]
            u2 = urows[b, pl.ds(2 * LANES, LANES)]
            u3 = urows[b, pl.ds(3 * LANES, LANES)]
            r0 = b * T
            for t in range(T):
                v0 = vrows[r0 + t, pl.ds(0, LANES)]
                v1 = vrows[r0 + t, pl.ds(LANES, LANES)]
                v2 = vrows[r0 + t, pl.ds(2 * LANES, LANES)]
                v3 = vrows[r0 + t, pl.ds(3 * LANES, LANES)]
                pbuf[r0 + t, :] = (u0 * v0 + u1 * v1) + (u2 * v2 + u3 * v3)

        pltpu.sync_copy(pbuf, out_hbm.at[pl.ds(b0 * T, WT)])


def _make_sc_scores():
    mesh = plsc.VectorSubcoreMesh(core_axis_name="c", subcore_axis_name="s")
    return pl.kernel(
        _sc_scores_body,
        out_type=jax.ShapeDtypeStruct((B * T, LANES), jnp.float32),
        mesh=mesh,
        compiler_params=pltpu.CompilerParams(use_tc_tiling_on_sc=False),
        scratch_types=[
            pltpu.VMEM((W,), jnp.int32),
            pltpu.VMEM((WT,), jnp.int32),
            pltpu.VMEM((W, D), jnp.float32),
            pltpu.VMEM((WT, D), jnp.float32),
            pltpu.VMEM((WT, LANES), jnp.float32),
            pltpu.SemaphoreType.DMA,
        ],
    )


_TC_ROWS = 400
_TC_COLS = 2048
_TC_GRID = (B * T * LANES) // (_TC_ROWS * _TC_COLS)
_DOTS_PER_ROW = _TC_COLS // LANES


def _tc_loss_body(x_ref, o_ref):
    i = pl.program_id(0)
    x = x_ref[...]
    j = lax.broadcasted_iota(jnp.int32, (_TC_COLS, _DOTS_PER_ROW), 0)
    d = lax.broadcasted_iota(jnp.int32, (_TC_COLS, _DOTS_PER_ROW), 1)
    m = (j // LANES == d).astype(jnp.float32)
    s = lax.dot_general(x, m, (((1,), (0,)), ((), ())),
                        preferred_element_type=jnp.float32,
                        precision=lax.Precision.HIGHEST)
    rr = lax.broadcasted_iota(jnp.int32, (_TC_ROWS, _DOTS_PER_ROW), 0)
    cc = lax.broadcasted_iota(jnp.int32, (_TC_ROWS, _DOTS_PER_ROW), 1)
    dot_id = (i * _TC_ROWS + rr) * _DOTS_PER_ROW + cc
    t = lax.rem(dot_id, T)
    sign = jnp.where(t < NPOS, 1.0, -1.0).astype(jnp.float32)
    z = sign * s
    val = jnp.minimum(z, 0.0) - jnp.log(1.0 + jnp.exp(-jnp.abs(z)))

    @pl.when(i == 0)
    def _():
        o_ref[0, 0] = 0.0

    o_ref[0, 0] -= jnp.sum(val)


def _make_tc_loss():
    return pl.pallas_call(
        _tc_loss_body,
        out_shape=jax.ShapeDtypeStruct((1, 1), jnp.float32),
        grid=(_TC_GRID,),
        in_specs=[pl.BlockSpec((_TC_ROWS, _TC_COLS), lambda i: (i, 0))],
        out_specs=pl.BlockSpec(block_shape=(1, 1), index_map=lambda i: (0, 0),
                               memory_space=pltpu.SMEM),
    )


def kernel(pos_u, pos_v, neg_v, u_weight, v_weight):
    pos_u = pos_u.astype(jnp.int32)
    tgt = jnp.concatenate([pos_v.astype(jnp.int32), neg_v.astype(jnp.int32)],
                          axis=1).reshape(B * T)
    partials = _make_sc_scores()(pos_u, tgt, u_weight, v_weight)
    p2 = partials.reshape((B * T * LANES) // _TC_COLS, _TC_COLS)
    loss = _make_tc_loss()(p2)
    return loss[0, 0]
